# Initial kernel scaffold; baseline (speedup 1.0000x reference)
import jax
import jax.numpy as jnp
from jax.experimental import pallas as pl


def kernel(x, edge_index, edge_weight, W1z, b1z, W1r, b1r, W1h, b1h, W2z, b2z, W2r, b2r, W2h, b2h, lin_W, lin_b):
    raise NotImplementedError("write your pallas kernel here")


# speedup vs baseline: 19.2821x; 19.2821x over previous
import functools

import jax
import jax.numpy as jnp
from jax import lax
from jax.experimental import pallas as pl
from jax.experimental.pallas import tpu as pltpu
from jax.experimental.pallas import tpu_sc as plsc

N_NODES = 10000
N_EDGES = 320000
D_FEAT = 128
H1 = 50
H2 = 20

NC = 2
NS = 16
NW = NC * NS
EPW = N_EDGES // NW
EPT = N_EDGES // NS
_NPAD = 10240
NPT = _NPAD // NS
ZR = 128


def _make_degrees():
  mesh = plsc.VectorSubcoreMesh(core_axis_name="c", subcore_axis_name="s")
  w = 1000
  nwin = EPW // w

  @functools.partial(
      pl.kernel,
      mesh=mesh,
      out_type=jax.ShapeDtypeStruct((NC, 2, _NPAD), jnp.float32),
      scratch_types=[
          pltpu.VMEM((w,), jnp.int32),
          pltpu.VMEM((w,), jnp.int32),
          pltpu.VMEM((w,), jnp.float32),
          pltpu.VMEM((NPT,), jnp.float32),
          pltpu.VMEM_SHARED((_NPAD,), jnp.float32),
          pltpu.VMEM_SHARED((_NPAD,), jnp.float32),
      ],
  )
  def degrees(row_hbm, col_hbm, ew_hbm, out_hbm, ridx_v, cidx_v, ew_v, zb_v,
              acc_o, acc_i):
    c = lax.axis_index("c")
    s = lax.axis_index("s")
    wid = s * NC + c

    zero16 = jnp.zeros((16,), jnp.float32)
    for j in range(NPT // 16):
      zb_v[pl.ds(j * 16, 16)] = zero16
    pltpu.sync_copy(zb_v, acc_o.at[pl.ds(s * NPT, NPT)])
    pltpu.sync_copy(zb_v, acc_i.at[pl.ds(s * NPT, NPT)])
    plsc.subcore_barrier()

    def win(wi, carry):
      base = wid * EPW + wi * w
      pltpu.sync_copy(row_hbm.at[pl.ds(base, w)], ridx_v)
      pltpu.sync_copy(col_hbm.at[pl.ds(base, w)], cidx_v)
      pltpu.sync_copy(ew_hbm.at[pl.ds(base, w)], ew_v)
      pltpu.sync_copy(ew_v, acc_o.at[ridx_v], add=True)
      pltpu.sync_copy(ew_v, acc_i.at[cidx_v], add=True)
      return carry

    lax.fori_loop(0, nwin, win, 0)
    plsc.subcore_barrier()
    pltpu.sync_copy(acc_o.at[pl.ds(s * NPT, NPT)],
                    out_hbm.at[c, 0, pl.ds(s * NPT, NPT)])
    pltpu.sync_copy(acc_i.at[pl.ds(s * NPT, NPT)],
                    out_hbm.at[c, 1, pl.ds(s * NPT, NPT)])

  return degrees


def _make_diffuse(dp, w):
  mesh = plsc.VectorSubcoreMesh(core_axis_name="c", subcore_axis_name="s")
  nwin = EPT // w
  assert w % 16 == 0 and EPT % w == 0

  @functools.partial(
      pl.kernel,
      mesh=mesh,
      compiler_params=pltpu.CompilerParams(use_tc_tiling_on_sc=False),
      out_type=jax.ShapeDtypeStruct((NC, _NPAD, dp), jnp.float32),
      scratch_types=[
          pltpu.VMEM((w,), jnp.int32),
          pltpu.VMEM((w,), jnp.int32),
          pltpu.VMEM((w,), jnp.float32),
          pltpu.VMEM((w, dp), jnp.float32),
          pltpu.VMEM_SHARED((_NPAD, dp), jnp.float32),
          pltpu.SemaphoreType.DMA,
      ],
  )
  def diffuse(u2_hbm, ei_hbm, ew_hbm, out_hbm,
              sidx_v, didx_v, ew_v, rows_v, acc_sh, sem):
    c = lax.axis_index("c")
    s = lax.axis_index("s")

    zero16 = jnp.zeros((16,), jnp.float32)

    def zrow(r, carry):
      for j in range(dp // 16):
        rows_v[r, pl.ds(j * 16, 16)] = zero16
      return carry

    lax.fori_loop(0, ZR, zrow, 0)
    for j in range(NPT // ZR):
      pltpu.sync_copy(rows_v.at[pl.ds(0, ZR)],
                      acc_sh.at[pl.ds(s * NPT + j * ZR, ZR)])
    plsc.subcore_barrier()

    def win(wi, carry):
      base = s * EPT + wi * w
      pltpu.sync_copy(ei_hbm.at[c, pl.ds(base, w)], sidx_v)
      pltpu.sync_copy(ei_hbm.at[1 - c, pl.ds(base, w)], didx_v)
      pltpu.sync_copy(ew_hbm.at[pl.ds(base, w)], ew_v)
      pltpu.async_copy(u2_hbm.at[c].at[sidx_v], rows_v, sem).wait()

      def blk16(i, bcarry):
        e0 = i * 16
        ewv = ew_v[pl.ds(e0, 16)]
        for k in range(16):
          ew16 = jnp.full((16,), ewv[k], jnp.float32)
          for j in range(dp // 16):
            rows_v[e0 + k, pl.ds(j * 16, 16)] = (
                rows_v[e0 + k, pl.ds(j * 16, 16)] * ew16)
        return bcarry

      lax.fori_loop(0, w // 16, blk16, 0)
      pltpu.sync_copy(rows_v, acc_sh.at[didx_v], add=True)
      return carry

    lax.fori_loop(0, nwin, win, 0)
    plsc.subcore_barrier()
    pltpu.sync_copy(acc_sh.at[pl.ds(s * NPT, NPT)],
                    out_hbm.at[c, pl.ds(s * NPT, NPT)])

  return diffuse


_degrees_call = _make_degrees()
_diffuse_112 = _make_diffuse(112, 400)
_diffuse_48 = _make_diffuse(48, 400)


_BLK = 640
_GRID = _NPAD // _BLK


def _inv_deg(deg_ref):
  d = deg_ref[...]
  do = d[:, 0] + d[:, 2]
  di = d[:, 1] + d[:, 3]
  invo = jnp.where(do == 0.0, 1.0, 1.0 / do)
  invi = jnp.where(di == 0.0, 1.0, 1.0 / di)
  return invo, invi


def _prep1_body(deg_ref, x_ref, w_ref, u2_ref, h0_ref):
  invo, invi = _inv_deg(deg_ref)
  p = jnp.dot(x_ref[...], w_ref[...], preferred_element_type=jnp.float32)
  u2_ref[0] = p[:, 0:112] * invo[:, None]
  u2_ref[1] = p[:, 112:224] * invi[:, None]
  h0_ref[...] = p[:, 224:336]


def _prep1(degp, x, wcat):
  return pl.pallas_call(
      _prep1_body,
      grid=(_GRID,),
      in_specs=[
          pl.BlockSpec((_BLK, 4), lambda i: (i, 0)),
          pl.BlockSpec((_BLK, D_FEAT), lambda i: (i, 0)),
          pl.BlockSpec((D_FEAT, 336), lambda i: (0, 0)),
      ],
      out_specs=[
          pl.BlockSpec((NC, _BLK, 112), lambda i: (0, i, 0)),
          pl.BlockSpec((_BLK, 112), lambda i: (i, 0)),
      ],
      out_shape=[
          jax.ShapeDtypeStruct((NC, _NPAD, 112), jnp.float32),
          jax.ShapeDtypeStruct((_NPAD, 112), jnp.float32),
      ],
  )(degp, x, wcat)


def _comb1_body(deg_ref, d_ref, h0_ref, bias_ref, w2_ref, u2_ref, h02_ref):
  invo, invi = _inv_deg(deg_ref)
  ssum = d_ref[0] + d_ref[1] + h0_ref[...] + bias_ref[...]
  hz = ssum[:, 0:H1]
  hh = ssum[:, H1:2 * H1]
  h1 = jnp.maximum(jnp.tanh(hh) / (1.0 + jnp.exp(hz)), 0.0)
  p2 = jnp.dot(h1, w2_ref[...], preferred_element_type=jnp.float32)
  u2_ref[0] = p2[:, 0:48] * invo[:, None]
  u2_ref[1] = p2[:, 48:96] * invi[:, None]
  h02_ref[...] = p2[:, 96:144]


def _comb1(degp, d1, h0, bias, w2cat):
  return pl.pallas_call(
      _comb1_body,
      grid=(_GRID,),
      in_specs=[
          pl.BlockSpec((_BLK, 4), lambda i: (i, 0)),
          pl.BlockSpec((NC, _BLK, 112), lambda i: (0, i, 0)),
          pl.BlockSpec((_BLK, 112), lambda i: (i, 0)),
          pl.BlockSpec((1, 112), lambda i: (0, 0)),
          pl.BlockSpec((H1, 144), lambda i: (0, 0)),
      ],
      out_specs=[
          pl.BlockSpec((NC, _BLK, 48), lambda i: (0, i, 0)),
          pl.BlockSpec((_BLK, 48), lambda i: (i, 0)),
      ],
      out_shape=[
          jax.ShapeDtypeStruct((NC, _NPAD, 48), jnp.float32),
          jax.ShapeDtypeStruct((_NPAD, 48), jnp.float32),
      ],
  )(degp, d1, h0, bias, w2cat)


def _comb2_body(d_ref, h0_ref, bias_ref, lw_ref, lb_ref, out_ref):
  ssum = d_ref[0] + d_ref[1] + h0_ref[...] + bias_ref[...]
  hz = ssum[:, 0:H2]
  hh = ssum[:, H2:2 * H2]
  h2 = jnp.maximum(jnp.tanh(hh) / (1.0 + jnp.exp(hz)), 0.0)
  out_ref[...] = (jnp.dot(h2, lw_ref[...], preferred_element_type=jnp.float32)
                  + lb_ref[...])


def _comb2(d2, h02, bias, lin_w, lin_b):
  return pl.pallas_call(
      _comb2_body,
      grid=(_GRID,),
      in_specs=[
          pl.BlockSpec((NC, _BLK, 48), lambda i: (0, i, 0)),
          pl.BlockSpec((_BLK, 48), lambda i: (i, 0)),
          pl.BlockSpec((1, 48), lambda i: (0, 0)),
          pl.BlockSpec((H2, 1), lambda i: (0, 0)),
          pl.BlockSpec((1, 1), lambda i: (0, 0)),
      ],
      out_specs=pl.BlockSpec((_BLK, 1), lambda i: (i, 0)),
      out_shape=jax.ShapeDtypeStruct((_NPAD, 1), jnp.float32),
  )(d2, h02, bias, lin_w, lin_b)


def _pad_cols(a, to):
  return jnp.pad(a, ((0, 0), (0, to - a.shape[1])))


def kernel(x, edge_index, edge_weight, W1z, b1z, W1r, b1r, W1h, b1h,
           W2z, b2z, W2r, b2r, W2h, b2h, lin_W, lin_b):
  row = edge_index[0]
  col = edge_index[1]

  w1z = W1z[:, :, :D_FEAT, :]
  w1h = W1h[:, :, :D_FEAT, :]
  wcat1 = jnp.concatenate([
      _pad_cols(jnp.concatenate([w1z[0, 1], w1h[0, 1]], axis=1), 112),
      _pad_cols(jnp.concatenate([w1z[1, 1], w1h[1, 1]], axis=1), 112),
      _pad_cols(jnp.concatenate([w1z[0, 0] + w1z[1, 0],
                                 w1h[0, 0] + w1h[1, 0]], axis=1), 112),
  ], axis=1)
  bias1 = _pad_cols(jnp.concatenate([b1z, b1h])[None, :], 112)

  w2z = W2z[:, :, :H1, :]
  w2h = W2h[:, :, :H1, :]
  w2cat = jnp.concatenate([
      _pad_cols(jnp.concatenate([w2z[0, 1], w2h[0, 1]], axis=1), 48),
      _pad_cols(jnp.concatenate([w2z[1, 1], w2h[1, 1]], axis=1), 48),
      _pad_cols(jnp.concatenate([w2z[0, 0] + w2z[1, 0],
                                 w2h[0, 0] + w2h[1, 0]], axis=1), 48),
  ], axis=1)
  bias2 = _pad_cols(jnp.concatenate([b2z, b2h])[None, :], 48)

  degp = _degrees_call(row, col, edge_weight)
  degp = degp.reshape(NC * 2, _NPAD).T

  x_pad = jnp.pad(x, ((0, _NPAD - N_NODES), (0, 0)))
  u1, h01 = _prep1(degp, x_pad, wcat1)
  d1 = _diffuse_112(u1, edge_index, edge_weight)
  u2, h02 = _comb1(degp, d1, h01, bias1, w2cat)
  d2 = _diffuse_48(u2, edge_index, edge_weight)
  out = _comb2(d2, h02, bias2, lin_W, lin_b[None, :])
  return out[:N_NODES]
